# baseline (speedup 1.0000x reference)
import functools

import jax
import jax.numpy as jnp
from jax import lax
from jax.experimental import pallas as pl
from jax.experimental.pallas import tpu as pltpu
from jax.experimental.pallas import tpu_sc as plsc

N_NODES = 10000
N_EDGES = 320000
D = 128
DH = 64

NP = 10240
NW = 32
ROWS_PER_SUB = NP // 16
CHUNK = 128
EPT = 10752
EP = NW * EPT
NCHUNK = EPT // CHUNK
PAD_NODE = 10200
BLK = 256
GRID = NP // BLK

_mesh = plsc.VectorSubcoreMesh(core_axis_name="c", subcore_axis_name="s")


@functools.partial(
    pl.kernel,
    out_type=jax.ShapeDtypeStruct((2 * NP, 16), jnp.float32),
    mesh=_mesh,
    scratch_types=[
        pltpu.VMEM_SHARED((NP, 16), jnp.float32),
        pltpu.VMEM((NCHUNK, CHUNK), jnp.int32),
        pltpu.VMEM((CHUNK, 16), jnp.float32),
    ],
    compiler_params=pltpu.CompilerParams(use_tc_tiling_on_sc=False),
)
def _deg_call(dst2_hbm, ones_hbm, zeros_hbm, out_hbm, dacc, didx, ones_v):
    cid = lax.axis_index("c")
    sid = lax.axis_index("s")
    wid = cid * 16 + sid
    rb = sid * ROWS_PER_SUB
    pltpu.sync_copy(zeros_hbm.at[pl.ds(rb, ROWS_PER_SUB)],
                    dacc.at[pl.ds(rb, ROWS_PER_SUB)])
    pltpu.sync_copy(ones_hbm, ones_v)
    pltpu.sync_copy(dst2_hbm.at[pl.ds(wid * NCHUNK, NCHUNK)], didx)
    plsc.subcore_barrier()

    def chunk(j, c):
        pltpu.sync_copy(ones_v, dacc.at[didx.at[j]], add=True)
        return c

    lax.fori_loop(0, NCHUNK, chunk, 0)
    plsc.subcore_barrier()
    pltpu.sync_copy(dacc.at[pl.ds(rb, ROWS_PER_SUB)],
                    out_hbm.at[pl.ds(cid * NP + rb, ROWS_PER_SUB)])


_mesh1 = plsc.VectorSubcoreMesh(core_axis_name="c", subcore_axis_name="s",
                                num_cores=1)


@functools.partial(
    pl.kernel,
    out_type=jax.ShapeDtypeStruct((NP, D), jnp.float32),
    mesh=_mesh1,
    scratch_types=[
        pltpu.VMEM_SHARED((NP, D), jnp.float32),
        [pltpu.VMEM((1, CHUNK), jnp.int32) for _ in range(2)],
        [pltpu.VMEM((1, CHUNK), jnp.int32) for _ in range(2)],
        [pltpu.VMEM((CHUNK, D), jnp.float32) for _ in range(2)],
        [pltpu.SemaphoreType.DMA for _ in range(2)],
    ],
    compiler_params=pltpu.CompilerParams(use_tc_tiling_on_sc=False),
)
def _prop_call(src2_hbm, dst2_hbm, h_hbm, zeros_hbm, out_hbm,
               acc, sidx, didx, rows, gsem):
    sid = lax.axis_index("s")
    rb = sid * ROWS_PER_SUB
    cbase = sid * NCHUNK
    pltpu.sync_copy(zeros_hbm.at[pl.ds(rb, ROWS_PER_SUB)],
                    acc.at[pl.ds(rb, ROWS_PER_SUB)])
    plsc.subcore_barrier()

    pltpu.sync_copy(src2_hbm.at[pl.ds(cbase, 1)], sidx[0])
    pltpu.sync_copy(dst2_hbm.at[pl.ds(cbase, 1)], didx[0])
    pltpu.async_copy(h_hbm.at[sidx[0].at[0]], rows[0], gsem[0])

    def pair(g, c):
        j0 = 2 * g
        pltpu.sync_copy(src2_hbm.at[pl.ds(cbase + j0 + 1, 1)], sidx[1])
        pltpu.sync_copy(dst2_hbm.at[pl.ds(cbase + j0 + 1, 1)], didx[1])
        pltpu.async_copy(h_hbm.at[sidx[1].at[0]], rows[1], gsem[1])
        pltpu.make_async_copy(h_hbm.at[sidx[0].at[0]], rows[0],
                              gsem[0]).wait()
        pltpu.sync_copy(rows[0], acc.at[didx[0].at[0]], add=True)

        @pl.when(j0 + 2 < NCHUNK)
        def _():
            pltpu.sync_copy(src2_hbm.at[pl.ds(cbase + j0 + 2, 1)], sidx[0])
            pltpu.sync_copy(dst2_hbm.at[pl.ds(cbase + j0 + 2, 1)], didx[0])
            pltpu.async_copy(h_hbm.at[sidx[0].at[0]], rows[0], gsem[0])

        pltpu.make_async_copy(h_hbm.at[sidx[1].at[0]], rows[1],
                              gsem[1]).wait()
        pltpu.sync_copy(rows[1], acc.at[didx[1].at[0]], add=True)
        return c

    lax.fori_loop(0, NCHUNK // 2, pair, 0)
    plsc.subcore_barrier()
    pltpu.sync_copy(acc.at[pl.ds(rb, ROWS_PER_SUB)],
                    out_hbm.at[pl.ds(rb, ROWS_PER_SUB)])


def _first_body(x_ref, w_ref, dp0_ref, dp1_ref, dinv_ref, hs_ref):
    deg = dp0_ref[:, :1] + dp1_ref[:, :1]
    dinv = jnp.where(deg > 0, lax.rsqrt(jnp.maximum(deg, 1e-12)), 0.0)
    dinvb = jnp.broadcast_to(dinv, (BLK, D))
    dinv_ref[...] = dinvb
    hs_ref[...] = dinvb * jnp.dot(x_ref[...], w_ref[...],
                                  preferred_element_type=jnp.float32)


def _first(x_pad, W1, dp):
    return pl.pallas_call(
        _first_body,
        grid=(GRID,),
        in_specs=[
            pl.BlockSpec((BLK, D), lambda i: (i, 0)),
            pl.BlockSpec((D, D), lambda i: (0, 0)),
            pl.BlockSpec((BLK, 16), lambda i: (i, 0)),
            pl.BlockSpec((BLK, 16), lambda i: (i + GRID, 0)),
        ],
        out_specs=[pl.BlockSpec((BLK, D), lambda i: (i, 0)),
                   pl.BlockSpec((BLK, D), lambda i: (i, 0))],
        out_shape=[jax.ShapeDtypeStruct((NP, D), jnp.float32),
                   jax.ShapeDtypeStruct((NP, D), jnp.float32)],
    )(x_pad, W1, dp, dp)


def _mix_body(p0_ref, p1_ref, dinv_ref, b_ref, w_ref, hs_ref):
    s = p0_ref[...] + p1_ref[...]
    xk = jnp.maximum(dinv_ref[...] * s + b_ref[...], 0.0)
    hs_ref[...] = dinv_ref[...] * jnp.dot(xk, w_ref[...],
                                          preferred_element_type=jnp.float32)


def _mix(p0, p1, dinv, b_prev, W):
    return pl.pallas_call(
        _mix_body,
        grid=(GRID,),
        in_specs=[
            pl.BlockSpec((BLK, D), lambda i: (i, 0)),
            pl.BlockSpec((BLK, D), lambda i: (i, 0)),
            pl.BlockSpec((BLK, D), lambda i: (i, 0)),
            pl.BlockSpec((1, D), lambda i: (0, 0)),
            pl.BlockSpec((D, D), lambda i: (0, 0)),
        ],
        out_specs=pl.BlockSpec((BLK, D), lambda i: (i, 0)),
        out_shape=jax.ShapeDtypeStruct((NP, D), jnp.float32),
    )(p0, p1, dinv, b_prev.reshape(1, D), W)


def _final_body(p0_ref, p1_ref, dinv_ref, b_ref, o_ref):
    o_ref[...] = dinv_ref[...] * (p0_ref[...] + p1_ref[...]) + b_ref[...]


def _final(p0, p1, dinv, b):
    return pl.pallas_call(
        _final_body,
        grid=(GRID,),
        in_specs=[
            pl.BlockSpec((BLK, D), lambda i: (i, 0)),
            pl.BlockSpec((BLK, D), lambda i: (i, 0)),
            pl.BlockSpec((BLK, D), lambda i: (i, 0)),
            pl.BlockSpec((1, D), lambda i: (0, 0)),
        ],
        out_specs=pl.BlockSpec((BLK, D), lambda i: (i, 0)),
        out_shape=jax.ShapeDtypeStruct((NP, D), jnp.float32),
    )(p0, p1, dinv, b.reshape(1, D))


def kernel(x, adj_t, W1, b1, W2, b2, W3, b3):
    loops = jnp.arange(N_NODES, dtype=jnp.int32)
    n_pad_e = EP - N_EDGES - N_NODES
    pad_e = jnp.full((n_pad_e,), PAD_NODE, dtype=jnp.int32)
    src = jnp.concatenate([adj_t[0].astype(jnp.int32), loops, pad_e])
    dst = jnp.concatenate([adj_t[1].astype(jnp.int32), loops, pad_e])
    src2 = src.reshape(EP // CHUNK, CHUNK)
    dst2 = dst.reshape(EP // CHUNK, CHUNK)

    x_pad = jnp.concatenate(
        [x, jnp.zeros((NP - N_NODES, D), jnp.float32)], axis=0)
    zeros_nd = jnp.zeros((NP, D), jnp.float32)
    zeros_n16 = jnp.zeros((NP, 16), jnp.float32)
    ones_16 = jnp.ones((CHUNK, 16), jnp.float32)

    nc2 = EP // CHUNK // 2
    src2a, src2b = src2[:nc2], src2[nc2:]
    dst2a, dst2b = dst2[:nc2], dst2[nc2:]

    dp = _deg_call(dst2, ones_16, zeros_n16)
    dinv, hs = _first(x_pad, W1, dp)
    for b_prev, W in ((b1, W2), (b2, W3)):
        p0 = _prop_call(src2a, dst2a, hs, zeros_nd)
        p1 = _prop_call(src2b, dst2b, hs, zeros_nd)
        hs = _mix(p0, p1, dinv, b_prev, W)
    p0 = _prop_call(src2a, dst2a, hs, zeros_nd)
    p1 = _prop_call(src2b, dst2b, hs, zeros_nd)
    out = _final(p0, p1, dinv, b3)
    return out[:N_NODES]


# speedup vs baseline: 1.2493x; 1.2493x over previous
import functools

import jax
import jax.numpy as jnp
from jax import lax
from jax.experimental import pallas as pl
from jax.experimental.pallas import tpu as pltpu
from jax.experimental.pallas import tpu_sc as plsc

N_NODES = 10000
N_EDGES = 320000
D = 128
DH = 64

NP = 10240
NW = 32
ROWS_PER_SUB = NP // 16
CHUNK = 128
EPT = 10752
EP = NW * EPT
NCHUNK = EPT // CHUNK
PAD_NODE = 10200
BLK = 256
GRID = NP // BLK

_mesh = plsc.VectorSubcoreMesh(core_axis_name="c", subcore_axis_name="s")


@functools.partial(
    pl.kernel,
    out_type=jax.ShapeDtypeStruct((2 * NP, 16), jnp.float32),
    mesh=_mesh,
    scratch_types=[
        pltpu.VMEM_SHARED((NP, 16), jnp.float32),
        pltpu.VMEM((NCHUNK, CHUNK), jnp.int32),
        pltpu.VMEM((CHUNK, 16), jnp.float32),
    ],
    compiler_params=pltpu.CompilerParams(use_tc_tiling_on_sc=False),
)
def _deg_call(dst2_hbm, ones_hbm, zeros_hbm, out_hbm, dacc, didx, ones_v):
    cid = lax.axis_index("c")
    sid = lax.axis_index("s")
    wid = cid * 16 + sid
    rb = sid * ROWS_PER_SUB
    pltpu.sync_copy(zeros_hbm.at[pl.ds(rb, ROWS_PER_SUB)],
                    dacc.at[pl.ds(rb, ROWS_PER_SUB)])
    pltpu.sync_copy(ones_hbm, ones_v)
    pltpu.sync_copy(dst2_hbm.at[pl.ds(wid * NCHUNK, NCHUNK)], didx)
    plsc.subcore_barrier()

    def chunk(j, c):
        pltpu.sync_copy(ones_v, dacc.at[didx.at[j]], add=True)
        return c

    lax.fori_loop(0, NCHUNK, chunk, 0)
    plsc.subcore_barrier()
    pltpu.sync_copy(dacc.at[pl.ds(rb, ROWS_PER_SUB)],
                    out_hbm.at[pl.ds(cid * NP + rb, ROWS_PER_SUB)])


@functools.partial(
    pl.kernel,
    out_type=jax.ShapeDtypeStruct((4 * NP, DH), jnp.float32),
    mesh=_mesh,
    scratch_types=[
        pltpu.VMEM_SHARED((NP, DH), jnp.float32),
        pltpu.VMEM((NCHUNK, CHUNK), jnp.int32),
        pltpu.VMEM((NCHUNK, CHUNK), jnp.int32),
        [pltpu.VMEM((CHUNK, DH), jnp.float32) for _ in range(6)],
        [pltpu.SemaphoreType.DMA for _ in range(6)],
        [pltpu.SemaphoreType.DMA for _ in range(6)],
    ],
    compiler_params=pltpu.CompilerParams(use_tc_tiling_on_sc=False),
)
def _prop_call(src2_hbm, dst2_hbm, ha_hbm, hb_hbm, zeros_hbm, out_hbm,
               acc, sidx, didx, rows, gsem, ssem):
    cid = lax.axis_index("c")
    sid = lax.axis_index("s")
    wid = cid * 16 + sid
    rb = sid * ROWS_PER_SUB
    cbase = wid * NCHUNK
    pltpu.sync_copy(src2_hbm.at[pl.ds(cbase, NCHUNK)], sidx)
    pltpu.sync_copy(dst2_hbm.at[pl.ds(cbase, NCHUNK)], didx)

    def run_pass(h_hbm, obase):
        pltpu.sync_copy(zeros_hbm.at[pl.ds(rb, ROWS_PER_SUB)],
                        acc.at[pl.ds(rb, ROWS_PER_SUB)])
        plsc.subcore_barrier()

        pltpu.async_copy(h_hbm.at[sidx.at[0]], rows[0], gsem[0])
        pltpu.async_copy(h_hbm.at[sidx.at[1]], rows[1], gsem[1])
        pltpu.async_copy(h_hbm.at[sidx.at[2]], rows[2], gsem[2])

        def group(g, c):
            j0 = 6 * g
            for k in range(6):
                j = j0 + k
                z = (k + 3) % 6

                @pl.when(j >= 3)
                def _():
                    pltpu.make_async_copy(
                        rows[z], acc.at[didx.at[0]], ssem[z]).wait()

                @pl.when(j + 3 < NCHUNK)
                def _():
                    pltpu.async_copy(h_hbm.at[sidx.at[j + 3]], rows[z],
                                     gsem[z])

                pltpu.make_async_copy(h_hbm.at[sidx.at[j]], rows[k],
                                      gsem[k]).wait()
                pltpu.async_copy(rows[k], acc.at[didx.at[j]], ssem[k],
                                 add=True)
            return c

        lax.fori_loop(0, NCHUNK // 6, group, 0)
        pltpu.make_async_copy(rows[3], acc.at[didx.at[0]], ssem[3]).wait()
        pltpu.make_async_copy(rows[4], acc.at[didx.at[0]], ssem[4]).wait()
        pltpu.make_async_copy(rows[5], acc.at[didx.at[0]], ssem[5]).wait()
        plsc.subcore_barrier()
        pltpu.sync_copy(acc.at[pl.ds(rb, ROWS_PER_SUB)],
                        out_hbm.at[pl.ds(obase + rb, ROWS_PER_SUB)])

    run_pass(ha_hbm, cid * 2 * NP)
    plsc.subcore_barrier()
    run_pass(hb_hbm, cid * 2 * NP + NP)


def _first_body(x_ref, w_ref, dp0_ref, dp1_ref, dinv_ref, hsa_ref, hsb_ref):
    deg = dp0_ref[:, :1] + dp1_ref[:, :1]
    dinv = jnp.where(deg > 0, lax.rsqrt(jnp.maximum(deg, 1e-12)), 0.0)
    dinvb = jnp.broadcast_to(dinv, (BLK, D))
    dinv_ref[...] = dinvb
    hs = dinvb * jnp.dot(x_ref[...], w_ref[...],
                         preferred_element_type=jnp.float32)
    hsa_ref[...] = hs[:, :DH]
    hsb_ref[...] = hs[:, DH:]


def _first(x_pad, W1, dp):
    return pl.pallas_call(
        _first_body,
        grid=(GRID,),
        in_specs=[
            pl.BlockSpec((BLK, D), lambda i: (i, 0)),
            pl.BlockSpec((D, D), lambda i: (0, 0)),
            pl.BlockSpec((BLK, 16), lambda i: (i, 0)),
            pl.BlockSpec((BLK, 16), lambda i: (i + GRID, 0)),
        ],
        out_specs=[pl.BlockSpec((BLK, D), lambda i: (i, 0)),
                   pl.BlockSpec((BLK, DH), lambda i: (i, 0)),
                   pl.BlockSpec((BLK, DH), lambda i: (i, 0))],
        out_shape=[jax.ShapeDtypeStruct((NP, D), jnp.float32),
                   jax.ShapeDtypeStruct((NP, DH), jnp.float32),
                   jax.ShapeDtypeStruct((NP, DH), jnp.float32)],
    )(x_pad, W1, dp, dp)


def _mix_body(pa0_ref, pa1_ref, pb0_ref, pb1_ref, dinv_ref, b_ref, w_ref,
              hsa_ref, hsb_ref):
    s = jnp.concatenate([pa0_ref[...] + pa1_ref[...],
                         pb0_ref[...] + pb1_ref[...]], axis=1)
    xk = jnp.maximum(dinv_ref[...] * s + b_ref[...], 0.0)
    hs = dinv_ref[...] * jnp.dot(xk, w_ref[...],
                                 preferred_element_type=jnp.float32)
    hsa_ref[...] = hs[:, :DH]
    hsb_ref[...] = hs[:, DH:]


def _mix(p, dinv, b_prev, W):
    return pl.pallas_call(
        _mix_body,
        grid=(GRID,),
        in_specs=[
            pl.BlockSpec((BLK, DH), lambda i: (i, 0)),
            pl.BlockSpec((BLK, DH), lambda i: (i + 2 * GRID, 0)),
            pl.BlockSpec((BLK, DH), lambda i: (i + GRID, 0)),
            pl.BlockSpec((BLK, DH), lambda i: (i + 3 * GRID, 0)),
            pl.BlockSpec((BLK, D), lambda i: (i, 0)),
            pl.BlockSpec((1, D), lambda i: (0, 0)),
            pl.BlockSpec((D, D), lambda i: (0, 0)),
        ],
        out_specs=[pl.BlockSpec((BLK, DH), lambda i: (i, 0)),
                   pl.BlockSpec((BLK, DH), lambda i: (i, 0))],
        out_shape=[jax.ShapeDtypeStruct((NP, DH), jnp.float32),
                   jax.ShapeDtypeStruct((NP, DH), jnp.float32)],
    )(p, p, p, p, dinv, b_prev.reshape(1, D), W)


def _final_body(pa0_ref, pa1_ref, pb0_ref, pb1_ref, dinv_ref, b_ref, o_ref):
    s = jnp.concatenate([pa0_ref[...] + pa1_ref[...],
                         pb0_ref[...] + pb1_ref[...]], axis=1)
    o_ref[...] = dinv_ref[...] * s + b_ref[...]


def _final(p, dinv, b):
    return pl.pallas_call(
        _final_body,
        grid=(GRID,),
        in_specs=[
            pl.BlockSpec((BLK, DH), lambda i: (i, 0)),
            pl.BlockSpec((BLK, DH), lambda i: (i + 2 * GRID, 0)),
            pl.BlockSpec((BLK, DH), lambda i: (i + GRID, 0)),
            pl.BlockSpec((BLK, DH), lambda i: (i + 3 * GRID, 0)),
            pl.BlockSpec((BLK, D), lambda i: (i, 0)),
            pl.BlockSpec((1, D), lambda i: (0, 0)),
        ],
        out_specs=pl.BlockSpec((BLK, D), lambda i: (i, 0)),
        out_shape=jax.ShapeDtypeStruct((NP, D), jnp.float32),
    )(p, p, p, p, dinv, b.reshape(1, D))


def kernel(x, adj_t, W1, b1, W2, b2, W3, b3):
    loops = jnp.arange(N_NODES, dtype=jnp.int32)
    n_pad_e = EP - N_EDGES - N_NODES
    pad_e = jnp.full((n_pad_e,), PAD_NODE, dtype=jnp.int32)
    src = jnp.concatenate([adj_t[0].astype(jnp.int32), loops, pad_e])
    dst = jnp.concatenate([adj_t[1].astype(jnp.int32), loops, pad_e])
    src2 = src.reshape(EP // CHUNK, CHUNK)
    dst2 = dst.reshape(EP // CHUNK, CHUNK)

    x_pad = jnp.concatenate(
        [x, jnp.zeros((NP - N_NODES, D), jnp.float32)], axis=0)
    zeros_nh = jnp.zeros((NP, DH), jnp.float32)
    zeros_n16 = jnp.zeros((NP, 16), jnp.float32)
    ones_16 = jnp.ones((CHUNK, 16), jnp.float32)

    dp = _deg_call(dst2, ones_16, zeros_n16)
    dinv, hsa, hsb = _first(x_pad, W1, dp)
    for b_prev, W in ((b1, W2), (b2, W3)):
        p = _prop_call(src2, dst2, hsa, hsb, zeros_nh)
        hsa, hsb = _mix(p, dinv, b_prev, W)
    p = _prop_call(src2, dst2, hsa, hsb, zeros_nh)
    out = _final(p, dinv, b3)
    return out[:N_NODES]


# speedup vs baseline: 5.0274x; 4.0240x over previous
import functools

import jax
import jax.numpy as jnp
from jax import lax
from jax.experimental import pallas as pl
from jax.experimental.pallas import tpu as pltpu
from jax.experimental.pallas import tpu_sc as plsc

N_NODES = 10000
N_EDGES = 320000
D = 128
DH = 64

NP = 10240
NW = 32
ROWS_PER_SUB = NP // 16
CHUNK = 128
EPT = 10752
EP = NW * EPT
NCHUNK = EPT // CHUNK
PAD_NODE = 10200
BLK = 256
GRID = NP // BLK

_mesh = plsc.VectorSubcoreMesh(core_axis_name="c", subcore_axis_name="s")


@functools.partial(
    pl.kernel,
    out_type=jax.ShapeDtypeStruct((2 * NP, 16), jnp.float32),
    mesh=_mesh,
    scratch_types=[
        pltpu.VMEM_SHARED((NP, 16), jnp.float32),
        pltpu.VMEM((NCHUNK, CHUNK), jnp.int32),
        pltpu.VMEM((CHUNK, 16), jnp.float32),
    ],
    compiler_params=pltpu.CompilerParams(use_tc_tiling_on_sc=False),
)
def _deg_call(dst2_hbm, ones_hbm, zeros_hbm, out_hbm, dacc, didx, ones_v):
    cid = lax.axis_index("c")
    sid = lax.axis_index("s")
    wid = cid * 16 + sid
    rb = sid * ROWS_PER_SUB
    pltpu.sync_copy(zeros_hbm.at[pl.ds(rb, ROWS_PER_SUB)],
                    dacc.at[pl.ds(rb, ROWS_PER_SUB)])
    pltpu.sync_copy(ones_hbm, ones_v)
    pltpu.sync_copy(dst2_hbm.at[pl.ds(wid * NCHUNK, NCHUNK)], didx)
    plsc.subcore_barrier()

    def chunk(j, c):
        pltpu.sync_copy(ones_v, dacc.at[didx.at[j]], add=True)
        return c

    lax.fori_loop(0, NCHUNK, chunk, 0)
    plsc.subcore_barrier()
    pltpu.sync_copy(dacc.at[pl.ds(rb, ROWS_PER_SUB)],
                    out_hbm.at[pl.ds(cid * NP + rb, ROWS_PER_SUB)])


@functools.partial(
    pl.kernel,
    out_type=jax.ShapeDtypeStruct((4 * NP, DH), jnp.float32),
    mesh=_mesh,
    scratch_types=[
        pltpu.VMEM_SHARED((NP, DH), jnp.float32),
        pltpu.VMEM((NCHUNK, CHUNK), jnp.int32),
        pltpu.VMEM((NCHUNK, CHUNK), jnp.int32),
        [pltpu.VMEM((CHUNK, DH), jnp.float32) for _ in range(6)],
        [pltpu.SemaphoreType.DMA for _ in range(6)],
        [pltpu.SemaphoreType.DMA for _ in range(6)],
    ],
    compiler_params=pltpu.CompilerParams(use_tc_tiling_on_sc=False),
)
def _prop_call(src2_hbm, dst2_hbm, ha_hbm, hb_hbm, zeros_hbm, out_hbm,
               acc, sidx, didx, rows, gsem, ssem):
    cid = lax.axis_index("c")
    sid = lax.axis_index("s")
    wid = cid * 16 + sid
    rb = sid * ROWS_PER_SUB
    cbase = wid * NCHUNK
    pltpu.sync_copy(src2_hbm.at[pl.ds(cbase, NCHUNK)], sidx)
    pltpu.sync_copy(dst2_hbm.at[pl.ds(cbase, NCHUNK)], didx)

    def run_pass(h_hbm, obase):
        pltpu.sync_copy(zeros_hbm.at[pl.ds(rb, ROWS_PER_SUB)],
                        acc.at[pl.ds(rb, ROWS_PER_SUB)])
        plsc.subcore_barrier()

        pltpu.async_copy(h_hbm.at[sidx.at[0]], rows[0], gsem[0])
        pltpu.async_copy(h_hbm.at[sidx.at[1]], rows[1], gsem[1])
        pltpu.async_copy(h_hbm.at[sidx.at[2]], rows[2], gsem[2])

        def group(g, c):
            j0 = 6 * g
            for k in range(6):
                j = j0 + k
                z = (k + 3) % 6

                @pl.when(j >= 3)
                def _():
                    pltpu.make_async_copy(
                        rows[z], acc.at[didx.at[0]], ssem[z]).wait()

                @pl.when(j + 3 < NCHUNK)
                def _():
                    pltpu.async_copy(h_hbm.at[sidx.at[j + 3]], rows[z],
                                     gsem[z])

                pltpu.make_async_copy(h_hbm.at[sidx.at[j]], rows[k],
                                      gsem[k]).wait()
                pltpu.async_copy(rows[k], acc.at[didx.at[j]], ssem[k],
                                 add=True)
            return c

        lax.fori_loop(0, NCHUNK // 6, group, 0)
        pltpu.make_async_copy(rows[3], acc.at[didx.at[0]], ssem[3]).wait()
        pltpu.make_async_copy(rows[4], acc.at[didx.at[0]], ssem[4]).wait()
        pltpu.make_async_copy(rows[5], acc.at[didx.at[0]], ssem[5]).wait()
        plsc.subcore_barrier()
        pltpu.sync_copy(acc.at[pl.ds(rb, ROWS_PER_SUB)],
                        out_hbm.at[pl.ds(obase + rb, ROWS_PER_SUB)])

    run_pass(ha_hbm, cid * 2 * NP)
    plsc.subcore_barrier()
    run_pass(hb_hbm, cid * 2 * NP + NP)


def _first_body(x_ref, w_ref, dp0_ref, dp1_ref, dinv_ref, hsa_ref, hsb_ref):
    deg = dp0_ref[:, :1] + dp1_ref[:, :1]
    dinv = jnp.where(deg > 0, lax.rsqrt(jnp.maximum(deg, 1e-12)), 0.0)
    dinvb = jnp.broadcast_to(dinv, (BLK, D))
    dinv_ref[...] = dinvb
    hs = dinvb * jnp.dot(x_ref[...], w_ref[...],
                         preferred_element_type=jnp.float32)
    hsa_ref[...] = hs[:, :DH]
    hsb_ref[...] = hs[:, DH:]


def _first(x_pad, W1, dp):
    return pl.pallas_call(
        _first_body,
        grid=(GRID,),
        in_specs=[
            pl.BlockSpec((BLK, D), lambda i: (i, 0)),
            pl.BlockSpec((D, D), lambda i: (0, 0)),
            pl.BlockSpec((BLK, 16), lambda i: (i, 0)),
            pl.BlockSpec((BLK, 16), lambda i: (i + GRID, 0)),
        ],
        out_specs=[pl.BlockSpec((BLK, D), lambda i: (i, 0)),
                   pl.BlockSpec((BLK, DH), lambda i: (i, 0)),
                   pl.BlockSpec((BLK, DH), lambda i: (i, 0))],
        out_shape=[jax.ShapeDtypeStruct((NP, D), jnp.float32),
                   jax.ShapeDtypeStruct((NP, DH), jnp.float32),
                   jax.ShapeDtypeStruct((NP, DH), jnp.float32)],
    )(x_pad, W1, dp, dp)


def _mix_body(pa0_ref, pa1_ref, pb0_ref, pb1_ref, dinv_ref, b_ref, w_ref,
              hsa_ref, hsb_ref):
    s = jnp.concatenate([pa0_ref[...] + pa1_ref[...],
                         pb0_ref[...] + pb1_ref[...]], axis=1)
    xk = jnp.maximum(dinv_ref[...] * s + b_ref[...], 0.0)
    hs = dinv_ref[...] * jnp.dot(xk, w_ref[...],
                                 preferred_element_type=jnp.float32)
    hsa_ref[...] = hs[:, :DH]
    hsb_ref[...] = hs[:, DH:]


def _mix(p, dinv, b_prev, W):
    return pl.pallas_call(
        _mix_body,
        grid=(GRID,),
        in_specs=[
            pl.BlockSpec((BLK, DH), lambda i: (i, 0)),
            pl.BlockSpec((BLK, DH), lambda i: (i + 2 * GRID, 0)),
            pl.BlockSpec((BLK, DH), lambda i: (i + GRID, 0)),
            pl.BlockSpec((BLK, DH), lambda i: (i + 3 * GRID, 0)),
            pl.BlockSpec((BLK, D), lambda i: (i, 0)),
            pl.BlockSpec((1, D), lambda i: (0, 0)),
            pl.BlockSpec((D, D), lambda i: (0, 0)),
        ],
        out_specs=[pl.BlockSpec((BLK, DH), lambda i: (i, 0)),
                   pl.BlockSpec((BLK, DH), lambda i: (i, 0))],
        out_shape=[jax.ShapeDtypeStruct((NP, DH), jnp.float32),
                   jax.ShapeDtypeStruct((NP, DH), jnp.float32)],
    )(p, p, p, p, dinv, b_prev.reshape(1, D), W)


def _final_body(pa0_ref, pa1_ref, pb0_ref, pb1_ref, dinv_ref, b_ref, o_ref):
    s = jnp.concatenate([pa0_ref[...] + pa1_ref[...],
                         pb0_ref[...] + pb1_ref[...]], axis=1)
    o_ref[...] = dinv_ref[...] * s + b_ref[...]


def _final(p, dinv, b):
    return pl.pallas_call(
        _final_body,
        grid=(GRID,),
        in_specs=[
            pl.BlockSpec((BLK, DH), lambda i: (i, 0)),
            pl.BlockSpec((BLK, DH), lambda i: (i + 2 * GRID, 0)),
            pl.BlockSpec((BLK, DH), lambda i: (i + GRID, 0)),
            pl.BlockSpec((BLK, DH), lambda i: (i + 3 * GRID, 0)),
            pl.BlockSpec((BLK, D), lambda i: (i, 0)),
            pl.BlockSpec((1, D), lambda i: (0, 0)),
        ],
        out_specs=pl.BlockSpec((BLK, D), lambda i: (i, 0)),
        out_shape=jax.ShapeDtypeStruct((NP, D), jnp.float32),
    )(p, p, p, p, dinv, b.reshape(1, D))


def kernel(x, adj_t, W1, b1, W2, b2, W3, b3):
    loops = jnp.arange(N_NODES, dtype=jnp.int32)
    n_pad_e = EP - N_EDGES - N_NODES
    pad_e = (jnp.arange(n_pad_e, dtype=jnp.int32) % (NP - N_NODES)) + N_NODES
    src = jnp.concatenate([adj_t[0].astype(jnp.int32), loops, pad_e])
    dst = jnp.concatenate([adj_t[1].astype(jnp.int32), loops, pad_e])
    src2 = src.reshape(EP // CHUNK, CHUNK)
    dst2 = dst.reshape(EP // CHUNK, CHUNK)

    x_pad = jnp.concatenate(
        [x, jnp.zeros((NP - N_NODES, D), jnp.float32)], axis=0)
    zeros_nh = jnp.zeros((NP, DH), jnp.float32)
    zeros_n16 = jnp.zeros((NP, 16), jnp.float32)
    ones_16 = jnp.ones((CHUNK, 16), jnp.float32)

    dp = _deg_call(dst2, ones_16, zeros_n16)
    dinv, hsa, hsb = _first(x_pad, W1, dp)
    for b_prev, W in ((b1, W2), (b2, W3)):
        p = _prop_call(src2, dst2, hsa, hsb, zeros_nh)
        hsa, hsb = _mix(p, dinv, b_prev, W)
    p = _prop_call(src2, dst2, hsa, hsb, zeros_nh)
    out = _final(p, dinv, b3)
    return out[:N_NODES]


# speedup vs baseline: 5.0288x; 1.0003x over previous
import functools

import jax
import jax.numpy as jnp
from jax import lax
from jax.experimental import pallas as pl
from jax.experimental.pallas import tpu as pltpu
from jax.experimental.pallas import tpu_sc as plsc

N_NODES = 10000
N_EDGES = 320000
D = 128
DH = 64

NP = 10240
NW = 32
ROWS_PER_SUB = NP // 16
CHUNK = 128
EPT = 10752
EP = NW * EPT
NCHUNK = EPT // CHUNK
BLK = 256
GRID = NP // BLK

_mesh = plsc.VectorSubcoreMesh(core_axis_name="c", subcore_axis_name="s")


@functools.partial(
    pl.kernel,
    out_type=jax.ShapeDtypeStruct((2 * NP, 16), jnp.float32),
    mesh=_mesh,
    scratch_types=[
        pltpu.VMEM_SHARED((NP, 16), jnp.float32),
        pltpu.VMEM((NCHUNK, CHUNK), jnp.int32),
        pltpu.VMEM((CHUNK, 16), jnp.float32),
    ],
    compiler_params=pltpu.CompilerParams(use_tc_tiling_on_sc=False),
)
def _deg_call(dst2_hbm, ones_hbm, zeros_hbm, out_hbm, dacc, didx, ones_v):
    cid = lax.axis_index("c")
    sid = lax.axis_index("s")
    wid = cid * 16 + sid
    rb = sid * ROWS_PER_SUB
    pltpu.sync_copy(zeros_hbm.at[pl.ds(rb, ROWS_PER_SUB)],
                    dacc.at[pl.ds(rb, ROWS_PER_SUB)])
    pltpu.sync_copy(ones_hbm, ones_v)
    pltpu.sync_copy(dst2_hbm.at[pl.ds(wid * NCHUNK, NCHUNK)], didx)
    plsc.subcore_barrier()

    def chunk(j, c):
        pltpu.sync_copy(ones_v, dacc.at[didx.at[j]], add=True)
        return c

    lax.fori_loop(0, NCHUNK, chunk, 0)
    plsc.subcore_barrier()
    pltpu.sync_copy(dacc.at[pl.ds(rb, ROWS_PER_SUB)],
                    out_hbm.at[pl.ds(cid * NP + rb, ROWS_PER_SUB)])


@functools.partial(
    pl.kernel,
    out_type=jax.ShapeDtypeStruct((4 * NP, DH), jnp.float32),
    mesh=_mesh,
    scratch_types=[
        pltpu.VMEM_SHARED((NP, DH), jnp.float32),
        pltpu.VMEM((NCHUNK, CHUNK), jnp.int32),
        pltpu.VMEM((NCHUNK, CHUNK), jnp.int32),
        [pltpu.VMEM((CHUNK, DH), jnp.float32) for _ in range(6)],
        [pltpu.SemaphoreType.DMA for _ in range(6)],
        [pltpu.SemaphoreType.DMA for _ in range(6)],
    ],
    compiler_params=pltpu.CompilerParams(use_tc_tiling_on_sc=False),
)
def _prop_call(src2_hbm, dst2_hbm, ha_hbm, hb_hbm, zeros_hbm, out_hbm,
               acc, sidx, didx, rows, gsem, ssem):
    cid = lax.axis_index("c")
    sid = lax.axis_index("s")
    wid = cid * 16 + sid
    rb = sid * ROWS_PER_SUB
    cbase = wid * NCHUNK
    pltpu.sync_copy(src2_hbm.at[pl.ds(cbase, NCHUNK)], sidx)
    pltpu.sync_copy(dst2_hbm.at[pl.ds(cbase, NCHUNK)], didx)

    def run_pass(h_hbm, obase):
        pltpu.sync_copy(zeros_hbm.at[pl.ds(rb, ROWS_PER_SUB)],
                        acc.at[pl.ds(rb, ROWS_PER_SUB)])
        plsc.subcore_barrier()

        pltpu.async_copy(h_hbm.at[sidx.at[0]], rows[0], gsem[0])
        pltpu.async_copy(h_hbm.at[sidx.at[1]], rows[1], gsem[1])
        pltpu.async_copy(h_hbm.at[sidx.at[2]], rows[2], gsem[2])

        def group(g, c):
            j0 = 6 * g
            for k in range(6):
                j = j0 + k
                z = (k + 3) % 6

                @pl.when(j >= 3)
                def _():
                    pltpu.make_async_copy(
                        rows[z], acc.at[didx.at[0]], ssem[z]).wait()

                @pl.when(j + 3 < NCHUNK)
                def _():
                    pltpu.async_copy(h_hbm.at[sidx.at[j + 3]], rows[z],
                                     gsem[z])

                pltpu.make_async_copy(h_hbm.at[sidx.at[j]], rows[k],
                                      gsem[k]).wait()
                pltpu.async_copy(rows[k], acc.at[didx.at[j]], ssem[k],
                                 add=True)
            return c

        lax.fori_loop(0, NCHUNK // 6, group, 0)
        pltpu.make_async_copy(rows[3], acc.at[didx.at[0]], ssem[3]).wait()
        pltpu.make_async_copy(rows[4], acc.at[didx.at[0]], ssem[4]).wait()
        pltpu.make_async_copy(rows[5], acc.at[didx.at[0]], ssem[5]).wait()
        plsc.subcore_barrier()
        pltpu.sync_copy(acc.at[pl.ds(rb, ROWS_PER_SUB)],
                        out_hbm.at[pl.ds(obase + rb, ROWS_PER_SUB)])

    run_pass(ha_hbm, cid * 2 * NP)
    plsc.subcore_barrier()
    run_pass(hb_hbm, cid * 2 * NP + NP)


def _first_body(x_ref, w_ref, dp0_ref, dp1_ref, dinv_ref, hsa_ref, hsb_ref):
    deg = dp0_ref[:, :1] + dp1_ref[:, :1]
    dinv = jnp.where(deg > 0, lax.rsqrt(jnp.maximum(deg, 1e-12)), 0.0)
    dinvb = jnp.broadcast_to(dinv, (BLK, D))
    dinv_ref[...] = dinvb
    hs = dinvb * jnp.dot(x_ref[...], w_ref[...],
                         preferred_element_type=jnp.float32)
    hsa_ref[...] = hs[:, :DH]
    hsb_ref[...] = hs[:, DH:]


def _first(x_pad, W1, dp):
    return pl.pallas_call(
        _first_body,
        grid=(GRID,),
        in_specs=[
            pl.BlockSpec((BLK, D), lambda i: (i, 0)),
            pl.BlockSpec((D, D), lambda i: (0, 0)),
            pl.BlockSpec((BLK, 16), lambda i: (i, 0)),
            pl.BlockSpec((BLK, 16), lambda i: (i + GRID, 0)),
        ],
        out_specs=[pl.BlockSpec((BLK, D), lambda i: (i, 0)),
                   pl.BlockSpec((BLK, DH), lambda i: (i, 0)),
                   pl.BlockSpec((BLK, DH), lambda i: (i, 0))],
        out_shape=[jax.ShapeDtypeStruct((NP, D), jnp.float32),
                   jax.ShapeDtypeStruct((NP, DH), jnp.float32),
                   jax.ShapeDtypeStruct((NP, DH), jnp.float32)],
    )(x_pad, W1, dp, dp)


def _mix_body(pa0_ref, pa1_ref, pb0_ref, pb1_ref, dinv_ref, b_ref, w_ref,
              hsa_ref, hsb_ref):
    s = jnp.concatenate([pa0_ref[...] + pa1_ref[...],
                         pb0_ref[...] + pb1_ref[...]], axis=1)
    xk = jnp.maximum(dinv_ref[...] * s + b_ref[...], 0.0)
    hs = dinv_ref[...] * jnp.dot(xk, w_ref[...],
                                 preferred_element_type=jnp.float32)
    hsa_ref[...] = hs[:, :DH]
    hsb_ref[...] = hs[:, DH:]


def _mix(p, dinv, b_prev, W):
    return pl.pallas_call(
        _mix_body,
        grid=(GRID,),
        in_specs=[
            pl.BlockSpec((BLK, DH), lambda i: (i, 0)),
            pl.BlockSpec((BLK, DH), lambda i: (i + 2 * GRID, 0)),
            pl.BlockSpec((BLK, DH), lambda i: (i + GRID, 0)),
            pl.BlockSpec((BLK, DH), lambda i: (i + 3 * GRID, 0)),
            pl.BlockSpec((BLK, D), lambda i: (i, 0)),
            pl.BlockSpec((1, D), lambda i: (0, 0)),
            pl.BlockSpec((D, D), lambda i: (0, 0)),
        ],
        out_specs=[pl.BlockSpec((BLK, DH), lambda i: (i, 0)),
                   pl.BlockSpec((BLK, DH), lambda i: (i, 0))],
        out_shape=[jax.ShapeDtypeStruct((NP, DH), jnp.float32),
                   jax.ShapeDtypeStruct((NP, DH), jnp.float32)],
    )(p, p, p, p, dinv, b_prev.reshape(1, D), W)


def _final_body(pa0_ref, pa1_ref, pb0_ref, pb1_ref, dinv_ref, b_ref, o_ref):
    s = jnp.concatenate([pa0_ref[...] + pa1_ref[...],
                         pb0_ref[...] + pb1_ref[...]], axis=1)
    o_ref[...] = dinv_ref[...] * s + b_ref[...]


def _final(p, dinv, b):
    return pl.pallas_call(
        _final_body,
        grid=(GRID,),
        in_specs=[
            pl.BlockSpec((BLK, DH), lambda i: (i, 0)),
            pl.BlockSpec((BLK, DH), lambda i: (i + 2 * GRID, 0)),
            pl.BlockSpec((BLK, DH), lambda i: (i + GRID, 0)),
            pl.BlockSpec((BLK, DH), lambda i: (i + 3 * GRID, 0)),
            pl.BlockSpec((BLK, D), lambda i: (i, 0)),
            pl.BlockSpec((1, D), lambda i: (0, 0)),
        ],
        out_specs=pl.BlockSpec((BLK, D), lambda i: (i, 0)),
        out_shape=jax.ShapeDtypeStruct((NP, D), jnp.float32),
    )(p, p, p, p, dinv, b.reshape(1, D))


def kernel(x, adj_t, W1, b1, W2, b2, W3, b3):
    loops = jnp.arange(N_NODES, dtype=jnp.int32)
    n_pad_e = EP - N_EDGES - N_NODES
    pad_e = (jnp.arange(n_pad_e, dtype=jnp.int32) % (NP - N_NODES)) + N_NODES
    src = jnp.concatenate([adj_t[0].astype(jnp.int32), loops, pad_e])
    dst = jnp.concatenate([adj_t[1].astype(jnp.int32), loops, pad_e])
    src2 = src.reshape(EP // CHUNK, CHUNK)
    dst2 = dst.reshape(EP // CHUNK, CHUNK)

    x_pad = jnp.concatenate(
        [x, jnp.zeros((NP - N_NODES, D), jnp.float32)], axis=0)
    zeros_nh = jnp.zeros((NP, DH), jnp.float32)
    zeros_n16 = jnp.zeros((NP, 16), jnp.float32)
    ones_16 = jnp.ones((CHUNK, 16), jnp.float32)

    dp = _deg_call(dst2, ones_16, zeros_n16)
    dinv, hsa, hsb = _first(x_pad, W1, dp)
    for b_prev, W in ((b1, W2), (b2, W3)):
        p = _prop_call(src2, dst2, hsa, hsb, zeros_nh)
        hsa, hsb = _mix(p, dinv, b_prev, W)
    p = _prop_call(src2, dst2, hsa, hsb, zeros_nh)
    out = _final(p, dinv, b3)
    return out[:N_NODES]
